# baseline (speedup 1.0000x reference)
import functools

import jax
import jax.numpy as jnp
from jax import lax
from jax.experimental import pallas as pl
from jax.experimental.pallas import tpu as pltpu
from jax.experimental.pallas import tpu_sc as plsc

N = 10000
E = 320000
F = 128
G = 64

NC = 2
NS = 16
NW = NC * NS

NPAD = 10240
CH = 128
EW = 10240
EPAD = NW * EW
NCHUNK = EW // CH
ROWS_T = NPAD // NS

BR = 1024
NBLK = NPAD // BR

_mesh = plsc.VectorSubcoreMesh(core_axis_name="c", subcore_axis_name="s")


@functools.partial(
    pl.kernel,
    out_type=jax.ShapeDtypeStruct((NC * NPAD,), jnp.float32),
    mesh=_mesh,
    scratch_types=[
        pltpu.VMEM((CH,), jnp.int32),
        pltpu.VMEM((CH,), jnp.float32),
        pltpu.VMEM_SHARED((NPAD,), jnp.float32),
    ],
)
def _deg_sc(dst_hbm, ones_hbm, zeros_hbm, out_hbm, didx, ones_v, dacc):
    c = lax.axis_index("c")
    s = lax.axis_index("s")
    wid = s * NC + c
    pltpu.sync_copy(zeros_hbm, dacc.at[pl.ds(s * ROWS_T, ROWS_T)])
    pltpu.sync_copy(ones_hbm, ones_v)
    plsc.subcore_barrier()
    base = wid * EW

    def body(i, _):
        off = pl.multiple_of(base + i * CH, CH)
        pltpu.sync_copy(dst_hbm.at[pl.ds(off, CH)], didx)
        pltpu.sync_copy(ones_v, dacc.at[didx], add=True)
        return ()

    lax.fori_loop(0, NCHUNK, body, ())
    plsc.subcore_barrier()
    pltpu.sync_copy(dacc.at[pl.ds(s * ROWS_T, ROWS_T)],
                    out_hbm.at[pl.ds(c * NPAD + s * ROWS_T, ROWS_T)])


NROW = 2
NIDX = 4
NCH_C0 = 160
NCH_C1 = 0
TOTCH = NW * NCHUNK


def _edge_pipeline(nch, base, src_hbm, dst_hbm, g_hbm,
                   sidx, didx, rows, acc, gsem, isem, ssem):
    for b in range(min(NIDX, nch)):
        pltpu.async_copy(src_hbm.at[base + b], sidx[b], isem[b])
        pltpu.async_copy(dst_hbm.at[base + b], didx[b], isem[b])

    def wait_idx(i, b):
        pltpu.make_async_copy(src_hbm.at[base + i], sidx[b], isem[b]).wait()
        pltpu.make_async_copy(dst_hbm.at[base + i], didx[b], isem[b]).wait()

    for k in range(min(NROW, nch)):
        wait_idx(k, k % NIDX)
        pltpu.async_copy(g_hbm.at[sidx[k % NIDX]], rows[k % NROW], gsem[k % NROW])

    def outer(g, _):
        for b in range(NIDX):
            i = g * NIDX + b
            rb = b % NROW
            pltpu.make_async_copy(g_hbm.at[sidx[b]], rows[rb],
                                  gsem[rb]).wait()
            pltpu.async_copy(rows[rb], acc.at[didx[b]], ssem,
                             add=True).wait()
            if nch > NIDX:
                @pl.when(i + NIDX < nch)
                def _():
                    pltpu.async_copy(src_hbm.at[base + i + NIDX], sidx[b],
                                     isem[b])
                    pltpu.async_copy(dst_hbm.at[base + i + NIDX], didx[b],
                                     isem[b])
            if nch > NROW:
                @pl.when(i + NROW < nch)
                def _():
                    b2 = (b + NROW) % NIDX
                    wait_idx(i + NROW, b2)
                    pltpu.async_copy(g_hbm.at[sidx[b2]], rows[rb],
                                     gsem[rb])
        return ()

    lax.fori_loop(0, nch // NIDX, outer, ())


@functools.partial(
    pl.kernel,
    out_type=jax.ShapeDtypeStruct((NPAD, F), jnp.float32),
    mesh=_mesh,
    scratch_types=[
        [pltpu.VMEM((CH,), jnp.int32) for _ in range(NIDX)],
        [pltpu.VMEM((CH,), jnp.int32) for _ in range(NIDX)],
        [pltpu.VMEM((CH, F), jnp.float32) for _ in range(NROW)],
        pltpu.VMEM_SHARED((NPAD, F), jnp.float32),
        [pltpu.SemaphoreType.DMA for _ in range(NROW)],
        [pltpu.SemaphoreType.DMA for _ in range(NIDX)],
        pltpu.SemaphoreType.DMA,
    ],
)
def _scatter_sc(src_hbm, dst_hbm, g_hbm, zeros_hbm, out_hbm,
                sidx, didx, rows, acc, gsem, isem, ssem):
    c = lax.axis_index("c")
    s = lax.axis_index("s")

    @pl.when(c == 0)
    def _():
        pltpu.sync_copy(zeros_hbm, acc.at[pl.ds(s * ROWS_T, ROWS_T)])
        plsc.subcore_barrier()
        _edge_pipeline(NCH_C0, s * NCH_C0, src_hbm, dst_hbm, g_hbm,
                       sidx, didx, rows, acc, gsem, isem, ssem)
        plsc.subcore_barrier()
        pltpu.sync_copy(acc.at[pl.ds(s * ROWS_T, ROWS_T)],
                        out_hbm.at[pl.ds(s * ROWS_T, ROWS_T)])


def _dinv(deg0_ref, deg1_ref):
    deg = deg0_ref[...] + deg1_ref[...] + 1.0
    return lax.rsqrt(deg)


def _first_tc_body(x_ref, deg0_ref, deg1_ref, w_ref, g_ref):
    h = jnp.dot(x_ref[...], w_ref[...], preferred_element_type=jnp.float32)
    g_ref[...] = _dinv(deg0_ref, deg1_ref) * h


def _mid_tc_body(a0_ref, gp_ref, deg0_ref, deg1_ref, b_ref, w_ref,
                 g_ref):
    dinv = _dinv(deg0_ref, deg1_ref)
    y = jnp.maximum(dinv * (a0_ref[...] + gp_ref[...])
                    + b_ref[...], 0.0)
    g_ref[...] = dinv * jnp.dot(y, w_ref[...],
                                preferred_element_type=jnp.float32)


def _final_tc_body(a0_ref, gp_ref, deg0_ref, deg1_ref, b_ref,
                   batch_ref, wout_ref, bout_ref, out_ref, sums, counts):
    i = pl.program_id(0)

    @pl.when(i == 0)
    def _():
        sums[...] = jnp.zeros_like(sums)
        counts[...] = jnp.zeros_like(counts)

    dinv = _dinv(deg0_ref, deg1_ref)
    y = jnp.maximum(dinv * (a0_ref[...] + gp_ref[...])
                    + b_ref[...], 0.0)
    oh = (jnp.broadcast_to(batch_ref[...].reshape(1, BR), (G, BR))
          == lax.broadcasted_iota(jnp.int32, (G, BR), 0)).astype(jnp.float32)
    sums[...] += jnp.dot(oh, y, preferred_element_type=jnp.float32)
    counts[...] += jnp.broadcast_to(
        jnp.sum(oh, axis=1, keepdims=True), (G, F))

    @pl.when(i == pl.num_programs(0) - 1)
    def _():
        pooled = sums[...] / jnp.maximum(counts[...], 1.0)
        out_ref[...] = (jnp.dot(pooled, wout_ref[...],
                                preferred_element_type=jnp.float32)
                        + bout_ref[...])


def _row_spec():
    return pl.BlockSpec((BR, F), lambda i: (i, 0))


def _deg_spec():
    return pl.BlockSpec((BR, 1), lambda i: (i, 0))


def _full_spec(r, c):
    return pl.BlockSpec((r, c), lambda i: (0, 0))


def _first_tc(xp, deg0, deg1, w):
    return pl.pallas_call(
        _first_tc_body,
        grid=(NBLK,),
        in_specs=[_row_spec(), _deg_spec(), _deg_spec(), _full_spec(F, F)],
        out_specs=_row_spec(),
        out_shape=jax.ShapeDtypeStruct((NPAD, F), jnp.float32),
    )(xp, deg0, deg1, w)


def _mid_tc(a0, gp, deg0, deg1, b2d, w):
    return pl.pallas_call(
        _mid_tc_body,
        grid=(NBLK,),
        in_specs=[_row_spec(), _row_spec(), _deg_spec(),
                  _deg_spec(), _full_spec(1, F), _full_spec(F, F)],
        out_specs=_row_spec(),
        out_shape=jax.ShapeDtypeStruct((NPAD, F), jnp.float32),
    )(a0, gp, deg0, deg1, b2d, w)


def _final_tc(a0, gp, deg0, deg1, b2d, batch2d, wout, bout2d):
    return pl.pallas_call(
        _final_tc_body,
        grid=(NBLK,),
        in_specs=[_row_spec(), _row_spec(), _deg_spec(),
                  _deg_spec(), _full_spec(1, F),
                  pl.BlockSpec((1, 1, BR), lambda i: (i, 0, 0)),
                  _full_spec(F, F), _full_spec(1, F)],
        out_specs=_full_spec(G, F),
        out_shape=jax.ShapeDtypeStruct((G, F), jnp.float32),
        scratch_shapes=[pltpu.VMEM((G, F), jnp.float32),
                        pltpu.VMEM((G, F), jnp.float32)],
    )(a0, gp, deg0, deg1, b2d, batch2d, wout, bout2d)


def kernel(x, edge_index, batch, W1, b1, W2, b2, W3, b3, Wout, bout):
    f32 = jnp.float32
    xp = jnp.concatenate([x, jnp.zeros((NPAD - N, F), f32)], axis=0)
    pad_e = jnp.full((EPAD - E,), NPAD - 1, jnp.int32)
    dst = jnp.concatenate([edge_index[1], pad_e])
    src2d = jnp.concatenate([edge_index[0], pad_e]).reshape(NW * NCHUNK, CH)
    dst2d = dst.reshape(NW * NCHUNK, CH)
    batch2d = jnp.concatenate(
        [batch.astype(jnp.int32), jnp.full((NPAD - N,), G, jnp.int32)]
    ).reshape(NBLK, 1, BR)

    zeros1 = jnp.zeros((ROWS_T,), f32)
    zeros128 = jnp.zeros((ROWS_T, F), f32)
    ones1 = jnp.ones((CH,), f32)

    degs = _deg_sc(dst, ones1, zeros1)
    deg0 = degs[:NPAD].reshape(NPAD, 1)
    deg1 = degs[NPAD:].reshape(NPAD, 1)

    b1r = b1.reshape(1, F)
    b2r = b2.reshape(1, F)
    b3r = b3.reshape(1, F)
    boutr = bout.reshape(1, F)

    g1 = _first_tc(xp, deg0, deg1, W1)
    a1 = _scatter_sc(src2d, dst2d, g1, zeros128)
    g2 = _mid_tc(a1, g1, deg0, deg1, b1r, W2)
    a2 = _scatter_sc(src2d, dst2d, g2, zeros128)
    g3 = _mid_tc(a2, g2, deg0, deg1, b2r, W3)
    a3 = _scatter_sc(src2d, dst2d, g3, zeros128)
    return _final_tc(a3, g3, deg0, deg1, b3r, batch2d, Wout, boutr)


# speedup vs baseline: 1.1117x; 1.1117x over previous
import functools

import jax
import jax.numpy as jnp
from jax import lax
from jax.experimental import pallas as pl
from jax.experimental.pallas import tpu as pltpu
from jax.experimental.pallas import tpu_sc as plsc

N = 10000
E = 320000
F = 128
G = 64

NC = 2
NS = 16
NW = NC * NS

NPAD = 10240
CH = 128
EW = 10240
EPAD = NW * EW
NCHUNK = EW // CH
ROWS_T = NPAD // NS

BR = 1024
NBLK = NPAD // BR

_mesh = plsc.VectorSubcoreMesh(core_axis_name="c", subcore_axis_name="s")


@functools.partial(
    pl.kernel,
    out_type=jax.ShapeDtypeStruct((NC * NPAD,), jnp.float32),
    mesh=_mesh,
    scratch_types=[
        pltpu.VMEM((CH,), jnp.int32),
        pltpu.VMEM((CH,), jnp.float32),
        pltpu.VMEM_SHARED((NPAD,), jnp.float32),
    ],
)
def _deg_sc(dst_hbm, ones_hbm, zeros_hbm, out_hbm, didx, ones_v, dacc):
    c = lax.axis_index("c")
    s = lax.axis_index("s")
    wid = s * NC + c
    pltpu.sync_copy(zeros_hbm, dacc.at[pl.ds(s * ROWS_T, ROWS_T)])
    pltpu.sync_copy(ones_hbm, ones_v)
    plsc.subcore_barrier()
    base = wid * EW

    def body(i, _):
        off = pl.multiple_of(base + i * CH, CH)
        pltpu.sync_copy(dst_hbm.at[pl.ds(off, CH)], didx)
        pltpu.sync_copy(ones_v, dacc.at[didx], add=True)
        return ()

    lax.fori_loop(0, NCHUNK, body, ())
    plsc.subcore_barrier()
    pltpu.sync_copy(dacc.at[pl.ds(s * ROWS_T, ROWS_T)],
                    out_hbm.at[pl.ds(c * NPAD + s * ROWS_T, ROWS_T)])


NROW = 2
NIDX = 4
NCH_C0 = 112
NCH_C1 = 48
TOTCH = NW * NCHUNK


def _edge_pipeline(nch, base, src_hbm, dst_hbm, g_hbm,
                   sidx, didx, rows, acc, gsem, isem, ssem):
    for b in range(min(NIDX, nch)):
        pltpu.async_copy(src_hbm.at[base + b], sidx[b], isem[b])
        pltpu.async_copy(dst_hbm.at[base + b], didx[b], isem[b])

    def wait_idx(i, b):
        pltpu.make_async_copy(src_hbm.at[base + i], sidx[b], isem[b]).wait()
        pltpu.make_async_copy(dst_hbm.at[base + i], didx[b], isem[b]).wait()

    for k in range(min(NROW, nch)):
        wait_idx(k, k % NIDX)
        pltpu.async_copy(g_hbm.at[sidx[k % NIDX]], rows[k % NROW], gsem[k % NROW])

    def outer(g, _):
        for b in range(NIDX):
            i = g * NIDX + b
            rb = b % NROW
            pltpu.make_async_copy(g_hbm.at[sidx[b]], rows[rb],
                                  gsem[rb]).wait()
            pltpu.async_copy(rows[rb], acc.at[didx[b]], ssem,
                             add=True).wait()
            if nch > NIDX:
                @pl.when(i + NIDX < nch)
                def _():
                    pltpu.async_copy(src_hbm.at[base + i + NIDX], sidx[b],
                                     isem[b])
                    pltpu.async_copy(dst_hbm.at[base + i + NIDX], didx[b],
                                     isem[b])
            if nch > NROW:
                @pl.when(i + NROW < nch)
                def _():
                    b2 = (b + NROW) % NIDX
                    wait_idx(i + NROW, b2)
                    pltpu.async_copy(g_hbm.at[sidx[b2]], rows[rb],
                                     gsem[rb])
        return ()

    lax.fori_loop(0, nch // NIDX, outer, ())


@functools.partial(
    pl.kernel,
    out_type=jax.ShapeDtypeStruct((NC, NPAD, F), jnp.float32),
    mesh=_mesh,
    scratch_types=[
        [pltpu.VMEM((CH,), jnp.int32) for _ in range(NIDX)],
        [pltpu.VMEM((CH,), jnp.int32) for _ in range(NIDX)],
        [pltpu.VMEM((CH, F), jnp.float32) for _ in range(NROW)],
        pltpu.VMEM_SHARED((NPAD, F), jnp.float32),
        [pltpu.SemaphoreType.DMA for _ in range(NROW)],
        [pltpu.SemaphoreType.DMA for _ in range(NIDX)],
        pltpu.SemaphoreType.DMA,
    ],
)
def _scatter_sc(src_hbm, dst_hbm, g_hbm, zeros_hbm, out_hbm,
                sidx, didx, rows, acc, gsem, isem, ssem):
    c = lax.axis_index("c")
    s = lax.axis_index("s")
    pltpu.sync_copy(zeros_hbm, acc.at[pl.ds(s * ROWS_T, ROWS_T)])
    plsc.subcore_barrier()

    @pl.when(c == 0)
    def _():
        _edge_pipeline(NCH_C0, s * NCH_C0, src_hbm, dst_hbm, g_hbm,
                       sidx, didx, rows, acc, gsem, isem, ssem)

    @pl.when(c == 1)
    def _():
        _edge_pipeline(NCH_C1, NS * NCH_C0 + s * NCH_C1, src_hbm, dst_hbm,
                       g_hbm, sidx, didx, rows, acc, gsem, isem, ssem)

    plsc.subcore_barrier()
    pltpu.sync_copy(acc.at[pl.ds(s * ROWS_T, ROWS_T)],
                    out_hbm.at[c, pl.ds(s * ROWS_T, ROWS_T)])


def _dinv(deg0_ref, deg1_ref):
    deg = deg0_ref[...] + deg1_ref[...] + 1.0
    return lax.rsqrt(deg)


def _first_tc_body(x_ref, deg0_ref, deg1_ref, w_ref, g_ref):
    h = jnp.dot(x_ref[...], w_ref[...], preferred_element_type=jnp.float32)
    g_ref[...] = _dinv(deg0_ref, deg1_ref) * h


def _mid_tc_body(a0_ref, a1_ref, gp_ref, deg0_ref, deg1_ref, b_ref, w_ref,
                 g_ref):
    dinv = _dinv(deg0_ref, deg1_ref)
    y = jnp.maximum(dinv * (a0_ref[...] + a1_ref[...] + gp_ref[...])
                    + b_ref[...], 0.0)
    g_ref[...] = dinv * jnp.dot(y, w_ref[...],
                                preferred_element_type=jnp.float32)


def _final_tc_body(a0_ref, a1_ref, gp_ref, deg0_ref, deg1_ref, b_ref,
                   batch_ref, wout_ref, bout_ref, out_ref, sums, counts):
    i = pl.program_id(0)

    @pl.when(i == 0)
    def _():
        sums[...] = jnp.zeros_like(sums)
        counts[...] = jnp.zeros_like(counts)

    dinv = _dinv(deg0_ref, deg1_ref)
    y = jnp.maximum(dinv * (a0_ref[...] + a1_ref[...] + gp_ref[...])
                    + b_ref[...], 0.0)
    oh = (jnp.broadcast_to(batch_ref[...].reshape(1, BR), (G, BR))
          == lax.broadcasted_iota(jnp.int32, (G, BR), 0)).astype(jnp.float32)
    sums[...] += jnp.dot(oh, y, preferred_element_type=jnp.float32)
    counts[...] += jnp.broadcast_to(
        jnp.sum(oh, axis=1, keepdims=True), (G, F))

    @pl.when(i == pl.num_programs(0) - 1)
    def _():
        pooled = sums[...] / jnp.maximum(counts[...], 1.0)
        out_ref[...] = (jnp.dot(pooled, wout_ref[...],
                                preferred_element_type=jnp.float32)
                        + bout_ref[...])


def _row_spec():
    return pl.BlockSpec((BR, F), lambda i: (i, 0))


def _deg_spec():
    return pl.BlockSpec((BR, 1), lambda i: (i, 0))


def _full_spec(r, c):
    return pl.BlockSpec((r, c), lambda i: (0, 0))


def _first_tc(xp, deg0, deg1, w):
    return pl.pallas_call(
        _first_tc_body,
        grid=(NBLK,),
        in_specs=[_row_spec(), _deg_spec(), _deg_spec(), _full_spec(F, F)],
        out_specs=_row_spec(),
        out_shape=jax.ShapeDtypeStruct((NPAD, F), jnp.float32),
    )(xp, deg0, deg1, w)


def _mid_tc(a0, a1, gp, deg0, deg1, b2d, w):
    return pl.pallas_call(
        _mid_tc_body,
        grid=(NBLK,),
        in_specs=[_row_spec(), _row_spec(), _row_spec(), _deg_spec(),
                  _deg_spec(), _full_spec(1, F), _full_spec(F, F)],
        out_specs=_row_spec(),
        out_shape=jax.ShapeDtypeStruct((NPAD, F), jnp.float32),
    )(a0, a1, gp, deg0, deg1, b2d, w)


def _final_tc(a0, a1, gp, deg0, deg1, b2d, batch2d, wout, bout2d):
    return pl.pallas_call(
        _final_tc_body,
        grid=(NBLK,),
        in_specs=[_row_spec(), _row_spec(), _row_spec(), _deg_spec(),
                  _deg_spec(), _full_spec(1, F),
                  pl.BlockSpec((1, 1, BR), lambda i: (i, 0, 0)),
                  _full_spec(F, F), _full_spec(1, F)],
        out_specs=_full_spec(G, F),
        out_shape=jax.ShapeDtypeStruct((G, F), jnp.float32),
        scratch_shapes=[pltpu.VMEM((G, F), jnp.float32),
                        pltpu.VMEM((G, F), jnp.float32)],
    )(a0, a1, gp, deg0, deg1, b2d, batch2d, wout, bout2d)


def kernel(x, edge_index, batch, W1, b1, W2, b2, W3, b3, Wout, bout):
    f32 = jnp.float32
    xp = jnp.concatenate([x, jnp.zeros((NPAD - N, F), f32)], axis=0)
    pad_e = jnp.full((EPAD - E,), NPAD - 1, jnp.int32)
    dst = jnp.concatenate([edge_index[1], pad_e])
    src2d = jnp.concatenate([edge_index[0], pad_e]).reshape(NW * NCHUNK, CH)
    dst2d = dst.reshape(NW * NCHUNK, CH)
    batch2d = jnp.concatenate(
        [batch.astype(jnp.int32), jnp.full((NPAD - N,), G, jnp.int32)]
    ).reshape(NBLK, 1, BR)

    zeros1 = jnp.zeros((ROWS_T,), f32)
    zeros128 = jnp.zeros((ROWS_T, F), f32)
    ones1 = jnp.ones((CH,), f32)

    degs = _deg_sc(dst, ones1, zeros1)
    deg0 = degs[:NPAD].reshape(NPAD, 1)
    deg1 = degs[NPAD:].reshape(NPAD, 1)

    b1r = b1.reshape(1, F)
    b2r = b2.reshape(1, F)
    b3r = b3.reshape(1, F)
    boutr = bout.reshape(1, F)

    g1 = _first_tc(xp, deg0, deg1, W1)
    a1 = _scatter_sc(src2d, dst2d, g1, zeros128)
    g2 = _mid_tc(a1[0], a1[1], g1, deg0, deg1, b1r, W2)
    a2 = _scatter_sc(src2d, dst2d, g2, zeros128)
    g3 = _mid_tc(a2[0], a2[1], g2, deg0, deg1, b2r, W3)
    a3 = _scatter_sc(src2d, dst2d, g3, zeros128)
    return _final_tc(a3[0], a3[1], g3, deg0, deg1, b3r, batch2d, Wout, boutr)


# speedup vs baseline: 1.1476x; 1.0323x over previous
import functools

import jax
import jax.numpy as jnp
from jax import lax
from jax.experimental import pallas as pl
from jax.experimental.pallas import tpu as pltpu
from jax.experimental.pallas import tpu_sc as plsc

N = 10000
E = 320000
F = 128
G = 64

NC = 2
NS = 16
NW = NC * NS

NPAD = 10240
CH = 128
EW = 10240
EPAD = NW * EW
NCHUNK = EW // CH
ROWS_T = NPAD // NS

BR = 1024
NBLK = NPAD // BR

_mesh = plsc.VectorSubcoreMesh(core_axis_name="c", subcore_axis_name="s")


@functools.partial(
    pl.kernel,
    out_type=jax.ShapeDtypeStruct((NC * NPAD,), jnp.float32),
    mesh=_mesh,
    scratch_types=[
        pltpu.VMEM((CH,), jnp.int32),
        pltpu.VMEM((CH,), jnp.float32),
        pltpu.VMEM_SHARED((NPAD,), jnp.float32),
    ],
)
def _deg_sc(dst_hbm, ones_hbm, zeros_hbm, out_hbm, didx, ones_v, dacc):
    c = lax.axis_index("c")
    s = lax.axis_index("s")
    wid = s * NC + c
    pltpu.sync_copy(zeros_hbm, dacc.at[pl.ds(s * ROWS_T, ROWS_T)])
    pltpu.sync_copy(ones_hbm, ones_v)
    plsc.subcore_barrier()
    base = wid * EW

    def body(i, _):
        off = pl.multiple_of(base + i * CH, CH)
        pltpu.sync_copy(dst_hbm.at[pl.ds(off, CH)], didx)
        pltpu.sync_copy(ones_v, dacc.at[didx], add=True)
        return ()

    lax.fori_loop(0, NCHUNK, body, ())
    plsc.subcore_barrier()
    pltpu.sync_copy(dacc.at[pl.ds(s * ROWS_T, ROWS_T)],
                    out_hbm.at[pl.ds(c * NPAD + s * ROWS_T, ROWS_T)])


NROW = 2
NIDX = 4
NCH_C0 = 136
NCH_C1 = 24
TOTCH = NW * NCHUNK


def _edge_pipeline(nch, base, src_hbm, dst_hbm, g_hbm,
                   sidx, didx, rows, acc, gsem, isem, ssem):
    for b in range(min(NIDX, nch)):
        pltpu.async_copy(src_hbm.at[base + b], sidx[b], isem[b])
        pltpu.async_copy(dst_hbm.at[base + b], didx[b], isem[b])

    def wait_idx(i, b):
        pltpu.make_async_copy(src_hbm.at[base + i], sidx[b], isem[b]).wait()
        pltpu.make_async_copy(dst_hbm.at[base + i], didx[b], isem[b]).wait()

    for k in range(min(NROW, nch)):
        wait_idx(k, k % NIDX)
        pltpu.async_copy(g_hbm.at[sidx[k % NIDX]], rows[k % NROW], gsem[k % NROW])

    def outer(g, _):
        for b in range(NIDX):
            i = g * NIDX + b
            rb = b % NROW
            pltpu.make_async_copy(g_hbm.at[sidx[b]], rows[rb],
                                  gsem[rb]).wait()
            pltpu.async_copy(rows[rb], acc.at[didx[b]], ssem,
                             add=True).wait()
            if nch > NIDX:
                @pl.when(i + NIDX < nch)
                def _():
                    pltpu.async_copy(src_hbm.at[base + i + NIDX], sidx[b],
                                     isem[b])
                    pltpu.async_copy(dst_hbm.at[base + i + NIDX], didx[b],
                                     isem[b])
            if nch > NROW:
                @pl.when(i + NROW < nch)
                def _():
                    b2 = (b + NROW) % NIDX
                    wait_idx(i + NROW, b2)
                    pltpu.async_copy(g_hbm.at[sidx[b2]], rows[rb],
                                     gsem[rb])
        return ()

    lax.fori_loop(0, nch // NIDX, outer, ())


@functools.partial(
    pl.kernel,
    out_type=jax.ShapeDtypeStruct((NC, NPAD, F), jnp.float32),
    mesh=_mesh,
    scratch_types=[
        [pltpu.VMEM((CH,), jnp.int32) for _ in range(NIDX)],
        [pltpu.VMEM((CH,), jnp.int32) for _ in range(NIDX)],
        [pltpu.VMEM((CH, F), jnp.float32) for _ in range(NROW)],
        pltpu.VMEM_SHARED((NPAD, F), jnp.float32),
        [pltpu.SemaphoreType.DMA for _ in range(NROW)],
        [pltpu.SemaphoreType.DMA for _ in range(NIDX)],
        pltpu.SemaphoreType.DMA,
    ],
)
def _scatter_sc(src_hbm, dst_hbm, g_hbm, zeros_hbm, out_hbm,
                sidx, didx, rows, acc, gsem, isem, ssem):
    c = lax.axis_index("c")
    s = lax.axis_index("s")
    pltpu.sync_copy(zeros_hbm, acc.at[pl.ds(s * ROWS_T, ROWS_T)])
    plsc.subcore_barrier()

    @pl.when(c == 0)
    def _():
        _edge_pipeline(NCH_C0, s * NCH_C0, src_hbm, dst_hbm, g_hbm,
                       sidx, didx, rows, acc, gsem, isem, ssem)

    @pl.when(c == 1)
    def _():
        _edge_pipeline(NCH_C1, NS * NCH_C0 + s * NCH_C1, src_hbm, dst_hbm,
                       g_hbm, sidx, didx, rows, acc, gsem, isem, ssem)

    plsc.subcore_barrier()
    pltpu.sync_copy(acc.at[pl.ds(s * ROWS_T, ROWS_T)],
                    out_hbm.at[c, pl.ds(s * ROWS_T, ROWS_T)])


def _dinv(deg0_ref, deg1_ref):
    deg = deg0_ref[...] + deg1_ref[...] + 1.0
    return lax.rsqrt(deg)


def _first_tc_body(x_ref, deg0_ref, deg1_ref, w_ref, g_ref):
    h = jnp.dot(x_ref[...], w_ref[...], preferred_element_type=jnp.float32)
    g_ref[...] = _dinv(deg0_ref, deg1_ref) * h


def _mid_tc_body(a0_ref, a1_ref, gp_ref, deg0_ref, deg1_ref, b_ref, w_ref,
                 g_ref):
    dinv = _dinv(deg0_ref, deg1_ref)
    y = jnp.maximum(dinv * (a0_ref[...] + a1_ref[...] + gp_ref[...])
                    + b_ref[...], 0.0)
    g_ref[...] = dinv * jnp.dot(y, w_ref[...],
                                preferred_element_type=jnp.float32)


def _final_tc_body(a0_ref, a1_ref, gp_ref, deg0_ref, deg1_ref, b_ref,
                   batch_ref, wout_ref, bout_ref, out_ref, sums, counts):
    i = pl.program_id(0)

    @pl.when(i == 0)
    def _():
        sums[...] = jnp.zeros_like(sums)
        counts[...] = jnp.zeros_like(counts)

    dinv = _dinv(deg0_ref, deg1_ref)
    y = jnp.maximum(dinv * (a0_ref[...] + a1_ref[...] + gp_ref[...])
                    + b_ref[...], 0.0)
    oh = (jnp.broadcast_to(batch_ref[...].reshape(1, BR), (G, BR))
          == lax.broadcasted_iota(jnp.int32, (G, BR), 0)).astype(jnp.float32)
    sums[...] += jnp.dot(oh, y, preferred_element_type=jnp.float32)
    counts[...] += jnp.broadcast_to(
        jnp.sum(oh, axis=1, keepdims=True), (G, F))

    @pl.when(i == pl.num_programs(0) - 1)
    def _():
        pooled = sums[...] / jnp.maximum(counts[...], 1.0)
        out_ref[...] = (jnp.dot(pooled, wout_ref[...],
                                preferred_element_type=jnp.float32)
                        + bout_ref[...])


def _row_spec():
    return pl.BlockSpec((BR, F), lambda i: (i, 0))


def _deg_spec():
    return pl.BlockSpec((BR, 1), lambda i: (i, 0))


def _full_spec(r, c):
    return pl.BlockSpec((r, c), lambda i: (0, 0))


def _first_tc(xp, deg0, deg1, w):
    return pl.pallas_call(
        _first_tc_body,
        grid=(NBLK,),
        in_specs=[_row_spec(), _deg_spec(), _deg_spec(), _full_spec(F, F)],
        out_specs=_row_spec(),
        out_shape=jax.ShapeDtypeStruct((NPAD, F), jnp.float32),
    )(xp, deg0, deg1, w)


def _mid_tc(a0, a1, gp, deg0, deg1, b2d, w):
    return pl.pallas_call(
        _mid_tc_body,
        grid=(NBLK,),
        in_specs=[_row_spec(), _row_spec(), _row_spec(), _deg_spec(),
                  _deg_spec(), _full_spec(1, F), _full_spec(F, F)],
        out_specs=_row_spec(),
        out_shape=jax.ShapeDtypeStruct((NPAD, F), jnp.float32),
    )(a0, a1, gp, deg0, deg1, b2d, w)


def _final_tc(a0, a1, gp, deg0, deg1, b2d, batch2d, wout, bout2d):
    return pl.pallas_call(
        _final_tc_body,
        grid=(NBLK,),
        in_specs=[_row_spec(), _row_spec(), _row_spec(), _deg_spec(),
                  _deg_spec(), _full_spec(1, F),
                  pl.BlockSpec((1, 1, BR), lambda i: (i, 0, 0)),
                  _full_spec(F, F), _full_spec(1, F)],
        out_specs=_full_spec(G, F),
        out_shape=jax.ShapeDtypeStruct((G, F), jnp.float32),
        scratch_shapes=[pltpu.VMEM((G, F), jnp.float32),
                        pltpu.VMEM((G, F), jnp.float32)],
    )(a0, a1, gp, deg0, deg1, b2d, batch2d, wout, bout2d)


def kernel(x, edge_index, batch, W1, b1, W2, b2, W3, b3, Wout, bout):
    f32 = jnp.float32
    xp = jnp.concatenate([x, jnp.zeros((NPAD - N, F), f32)], axis=0)
    pad_e = jnp.full((EPAD - E,), NPAD - 1, jnp.int32)
    dst = jnp.concatenate([edge_index[1], pad_e])
    src2d = jnp.concatenate([edge_index[0], pad_e]).reshape(NW * NCHUNK, CH)
    dst2d = dst.reshape(NW * NCHUNK, CH)
    batch2d = jnp.concatenate(
        [batch.astype(jnp.int32), jnp.full((NPAD - N,), G, jnp.int32)]
    ).reshape(NBLK, 1, BR)

    zeros1 = jnp.zeros((ROWS_T,), f32)
    zeros128 = jnp.zeros((ROWS_T, F), f32)
    ones1 = jnp.ones((CH,), f32)

    degs = _deg_sc(dst, ones1, zeros1)
    deg0 = degs[:NPAD].reshape(NPAD, 1)
    deg1 = degs[NPAD:].reshape(NPAD, 1)

    b1r = b1.reshape(1, F)
    b2r = b2.reshape(1, F)
    b3r = b3.reshape(1, F)
    boutr = bout.reshape(1, F)

    g1 = _first_tc(xp, deg0, deg1, W1)
    a1 = _scatter_sc(src2d, dst2d, g1, zeros128)
    g2 = _mid_tc(a1[0], a1[1], g1, deg0, deg1, b1r, W2)
    a2 = _scatter_sc(src2d, dst2d, g2, zeros128)
    g3 = _mid_tc(a2[0], a2[1], g2, deg0, deg1, b2r, W3)
    a3 = _scatter_sc(src2d, dst2d, g3, zeros128)
    return _final_tc(a3[0], a3[1], g3, deg0, deg1, b3r, batch2d, Wout, boutr)


# speedup vs baseline: 1.2808x; 1.1161x over previous
import functools

import jax
import jax.numpy as jnp
from jax import lax
from jax.experimental import pallas as pl
from jax.experimental.pallas import tpu as pltpu
from jax.experimental.pallas import tpu_sc as plsc

N = 10000
E = 320000
F = 128
G = 64

NC = 2
NS = 16
NW = NC * NS

NPAD = 10240
CH = 128
EW = 10240
EPAD = NW * EW
NCHUNK = EW // CH
ROWS_T = NPAD // NS

BR = 1024
NBLK = NPAD // BR

_mesh = plsc.VectorSubcoreMesh(core_axis_name="c", subcore_axis_name="s")


@functools.partial(
    pl.kernel,
    out_type=jax.ShapeDtypeStruct((NC * NPAD,), jnp.float32),
    mesh=_mesh,
    scratch_types=[
        pltpu.VMEM((CH,), jnp.int32),
        pltpu.VMEM((CH,), jnp.float32),
        pltpu.VMEM_SHARED((NPAD,), jnp.float32),
    ],
)
def _deg_sc(dst_hbm, ones_hbm, zeros_hbm, out_hbm, didx, ones_v, dacc):
    c = lax.axis_index("c")
    s = lax.axis_index("s")
    wid = s * NC + c
    pltpu.sync_copy(zeros_hbm, dacc.at[pl.ds(s * ROWS_T, ROWS_T)])
    pltpu.sync_copy(ones_hbm, ones_v)
    plsc.subcore_barrier()
    base = wid * EW

    def body(i, _):
        off = pl.multiple_of(base + i * CH, CH)
        pltpu.sync_copy(dst_hbm.at[pl.ds(off, CH)], didx)
        pltpu.sync_copy(ones_v, dacc.at[didx], add=True)
        return ()

    lax.fori_loop(0, NCHUNK, body, ())
    plsc.subcore_barrier()
    pltpu.sync_copy(dacc.at[pl.ds(s * ROWS_T, ROWS_T)],
                    out_hbm.at[pl.ds(c * NPAD + s * ROWS_T, ROWS_T)])


NROW = 2
NIDX = 4
NCH_C0 = 144
NCH_C1 = 16
TOTCH = NW * NCHUNK


def _edge_pipeline(nch, base, src_hbm, dst_hbm, g_hbm,
                   sidx, didx, rows, acc, gsem, isem, ssem):
    for b in range(min(NIDX, nch)):
        pltpu.async_copy(src_hbm.at[base + b], sidx[b], isem[b])
        pltpu.async_copy(dst_hbm.at[base + b], didx[b], isem[b])

    def wait_idx(i, b):
        pltpu.make_async_copy(src_hbm.at[base + i], sidx[b], isem[b]).wait()
        pltpu.make_async_copy(dst_hbm.at[base + i], didx[b], isem[b]).wait()

    for k in range(min(NROW, nch)):
        wait_idx(k, k % NIDX)
        pltpu.async_copy(g_hbm.at[sidx[k % NIDX]], rows[k % NROW], gsem[k % NROW])

    def outer(g, _):
        for b in range(NIDX):
            i = g * NIDX + b
            rb = b % NROW
            pltpu.make_async_copy(g_hbm.at[sidx[b]], rows[rb],
                                  gsem[rb]).wait()
            pltpu.async_copy(rows[rb], acc.at[didx[b]], ssem,
                             add=True).wait()
            if nch > NIDX:
                @pl.when(i + NIDX < nch)
                def _():
                    pltpu.async_copy(src_hbm.at[base + i + NIDX], sidx[b],
                                     isem[b])
                    pltpu.async_copy(dst_hbm.at[base + i + NIDX], didx[b],
                                     isem[b])
            if nch > NROW:
                @pl.when(i + NROW < nch)
                def _():
                    b2 = (b + NROW) % NIDX
                    wait_idx(i + NROW, b2)
                    pltpu.async_copy(g_hbm.at[sidx[b2]], rows[rb],
                                     gsem[rb])
        return ()

    lax.fori_loop(0, nch // NIDX, outer, ())


@functools.partial(
    pl.kernel,
    out_type=jax.ShapeDtypeStruct((NC, NPAD, F), jnp.float32),
    mesh=_mesh,
    scratch_types=[
        [pltpu.VMEM((CH,), jnp.int32) for _ in range(NIDX)],
        [pltpu.VMEM((CH,), jnp.int32) for _ in range(NIDX)],
        [pltpu.VMEM((CH, F), jnp.float32) for _ in range(NROW)],
        pltpu.VMEM_SHARED((NPAD, F), jnp.float32),
        [pltpu.SemaphoreType.DMA for _ in range(NROW)],
        [pltpu.SemaphoreType.DMA for _ in range(NIDX)],
        pltpu.SemaphoreType.DMA,
    ],
)
def _scatter_sc(src_hbm, dst_hbm, g_hbm, zeros_hbm, out_hbm,
                sidx, didx, rows, acc, gsem, isem, ssem):
    c = lax.axis_index("c")
    s = lax.axis_index("s")
    pltpu.sync_copy(zeros_hbm, acc.at[pl.ds(s * ROWS_T, ROWS_T)])
    plsc.subcore_barrier()

    @pl.when(c == 0)
    def _():
        _edge_pipeline(NCH_C0, s * NCH_C0, src_hbm, dst_hbm, g_hbm,
                       sidx, didx, rows, acc, gsem, isem, ssem)

    @pl.when(c == 1)
    def _():
        _edge_pipeline(NCH_C1, NS * NCH_C0 + s * NCH_C1, src_hbm, dst_hbm,
                       g_hbm, sidx, didx, rows, acc, gsem, isem, ssem)

    plsc.subcore_barrier()
    pltpu.sync_copy(acc.at[pl.ds(s * ROWS_T, ROWS_T)],
                    out_hbm.at[c, pl.ds(s * ROWS_T, ROWS_T)])


def _dinv(deg0_ref, deg1_ref):
    deg = deg0_ref[...] + deg1_ref[...] + 1.0
    return lax.rsqrt(deg)


def _first_tc_body(x_ref, deg0_ref, deg1_ref, w_ref, g_ref):
    h = jnp.dot(x_ref[...], w_ref[...], preferred_element_type=jnp.float32)
    g_ref[...] = _dinv(deg0_ref, deg1_ref) * h


def _mid_tc_body(a0_ref, a1_ref, gp_ref, deg0_ref, deg1_ref, b_ref, w_ref,
                 g_ref):
    dinv = _dinv(deg0_ref, deg1_ref)
    y = jnp.maximum(dinv * (a0_ref[...] + a1_ref[...] + gp_ref[...])
                    + b_ref[...], 0.0)
    g_ref[...] = dinv * jnp.dot(y, w_ref[...],
                                preferred_element_type=jnp.float32)


def _final_tc_body(a0_ref, a1_ref, gp_ref, deg0_ref, deg1_ref, b_ref,
                   batch_ref, wout_ref, bout_ref, out_ref, sums, counts):
    i = pl.program_id(0)

    @pl.when(i == 0)
    def _():
        sums[...] = jnp.zeros_like(sums)
        counts[...] = jnp.zeros_like(counts)

    dinv = _dinv(deg0_ref, deg1_ref)
    y = jnp.maximum(dinv * (a0_ref[...] + a1_ref[...] + gp_ref[...])
                    + b_ref[...], 0.0)
    oh = (jnp.broadcast_to(batch_ref[...].reshape(1, BR), (G, BR))
          == lax.broadcasted_iota(jnp.int32, (G, BR), 0)).astype(jnp.float32)
    sums[...] += jnp.dot(oh, y, preferred_element_type=jnp.float32)
    counts[...] += jnp.broadcast_to(
        jnp.sum(oh, axis=1, keepdims=True), (G, F))

    @pl.when(i == pl.num_programs(0) - 1)
    def _():
        pooled = sums[...] / jnp.maximum(counts[...], 1.0)
        out_ref[...] = (jnp.dot(pooled, wout_ref[...],
                                preferred_element_type=jnp.float32)
                        + bout_ref[...])


def _row_spec():
    return pl.BlockSpec((BR, F), lambda i: (i, 0))


def _deg_spec():
    return pl.BlockSpec((BR, 1), lambda i: (i, 0))


def _full_spec(r, c):
    return pl.BlockSpec((r, c), lambda i: (0, 0))


def _first_tc(xp, deg0, deg1, w):
    return pl.pallas_call(
        _first_tc_body,
        grid=(NBLK,),
        in_specs=[_row_spec(), _deg_spec(), _deg_spec(), _full_spec(F, F)],
        out_specs=_row_spec(),
        out_shape=jax.ShapeDtypeStruct((NPAD, F), jnp.float32),
    )(xp, deg0, deg1, w)


def _mid_tc(a0, a1, gp, deg0, deg1, b2d, w):
    return pl.pallas_call(
        _mid_tc_body,
        grid=(NBLK,),
        in_specs=[_row_spec(), _row_spec(), _row_spec(), _deg_spec(),
                  _deg_spec(), _full_spec(1, F), _full_spec(F, F)],
        out_specs=_row_spec(),
        out_shape=jax.ShapeDtypeStruct((NPAD, F), jnp.float32),
    )(a0, a1, gp, deg0, deg1, b2d, w)


def _final_tc(a0, a1, gp, deg0, deg1, b2d, batch2d, wout, bout2d):
    return pl.pallas_call(
        _final_tc_body,
        grid=(NBLK,),
        in_specs=[_row_spec(), _row_spec(), _row_spec(), _deg_spec(),
                  _deg_spec(), _full_spec(1, F),
                  pl.BlockSpec((1, 1, BR), lambda i: (i, 0, 0)),
                  _full_spec(F, F), _full_spec(1, F)],
        out_specs=_full_spec(G, F),
        out_shape=jax.ShapeDtypeStruct((G, F), jnp.float32),
        scratch_shapes=[pltpu.VMEM((G, F), jnp.float32),
                        pltpu.VMEM((G, F), jnp.float32)],
    )(a0, a1, gp, deg0, deg1, b2d, batch2d, wout, bout2d)


def kernel(x, edge_index, batch, W1, b1, W2, b2, W3, b3, Wout, bout):
    f32 = jnp.float32
    xp = jnp.concatenate([x, jnp.zeros((NPAD - N, F), f32)], axis=0)
    pad_e = jnp.full((EPAD - E,), NPAD - 1, jnp.int32)
    dst = jnp.concatenate([edge_index[1], pad_e])
    src2d = jnp.concatenate([edge_index[0], pad_e]).reshape(NW * NCHUNK, CH)
    dst2d = dst.reshape(NW * NCHUNK, CH)
    batch2d = jnp.concatenate(
        [batch.astype(jnp.int32), jnp.full((NPAD - N,), G, jnp.int32)]
    ).reshape(NBLK, 1, BR)

    zeros1 = jnp.zeros((ROWS_T,), f32)
    zeros128 = jnp.zeros((ROWS_T, F), f32)
    ones1 = jnp.ones((CH,), f32)

    degs = _deg_sc(dst, ones1, zeros1)
    deg0 = degs[:NPAD].reshape(NPAD, 1)
    deg1 = degs[NPAD:].reshape(NPAD, 1)

    b1r = b1.reshape(1, F)
    b2r = b2.reshape(1, F)
    b3r = b3.reshape(1, F)
    boutr = bout.reshape(1, F)

    g1 = _first_tc(xp, deg0, deg1, W1)
    a1 = _scatter_sc(src2d, dst2d, g1, zeros128)
    g2 = _mid_tc(a1[0], a1[1], g1, deg0, deg1, b1r, W2)
    a2 = _scatter_sc(src2d, dst2d, g2, zeros128)
    g3 = _mid_tc(a2[0], a2[1], g2, deg0, deg1, b2r, W3)
    a3 = _scatter_sc(src2d, dst2d, g3, zeros128)
    return _final_tc(a3[0], a3[1], g3, deg0, deg1, b3r, batch2d, Wout, boutr)


# speedup vs baseline: 1.2944x; 1.0106x over previous
import functools

import jax
import jax.numpy as jnp
from jax import lax
from jax.experimental import pallas as pl
from jax.experimental.pallas import tpu as pltpu
from jax.experimental.pallas import tpu_sc as plsc

N = 10000
E = 320000
F = 128
G = 64

NC = 2
NS = 16
NW = NC * NS

NPAD = 10240
CH = 128
EW = 10240
EPAD = NW * EW
NCHUNK = EW // CH
ROWS_T = NPAD // NS

BR = 1024
NBLK = NPAD // BR

_mesh = plsc.VectorSubcoreMesh(core_axis_name="c", subcore_axis_name="s")


@functools.partial(
    pl.kernel,
    out_type=jax.ShapeDtypeStruct((NC * NPAD,), jnp.float32),
    mesh=_mesh,
    scratch_types=[
        pltpu.VMEM((CH,), jnp.int32),
        pltpu.VMEM((CH,), jnp.float32),
        pltpu.VMEM_SHARED((NPAD,), jnp.float32),
    ],
)
def _deg_sc(dst_hbm, ones_hbm, zeros_hbm, out_hbm, didx, ones_v, dacc):
    c = lax.axis_index("c")
    s = lax.axis_index("s")
    wid = s * NC + c
    pltpu.sync_copy(zeros_hbm, dacc.at[pl.ds(s * ROWS_T, ROWS_T)])
    pltpu.sync_copy(ones_hbm, ones_v)
    plsc.subcore_barrier()
    base = wid * EW

    def body(i, _):
        off = pl.multiple_of(base + i * CH, CH)
        pltpu.sync_copy(dst_hbm.at[pl.ds(off, CH)], didx)
        pltpu.sync_copy(ones_v, dacc.at[didx], add=True)
        return ()

    lax.fori_loop(0, NCHUNK, body, ())
    plsc.subcore_barrier()
    pltpu.sync_copy(dacc.at[pl.ds(s * ROWS_T, ROWS_T)],
                    out_hbm.at[pl.ds(c * NPAD + s * ROWS_T, ROWS_T)])


NROW = 2
NIDX = 4
NCH_C0 = 152
NCH_C1 = 8
TOTCH = NW * NCHUNK


def _edge_pipeline(nch, base, src_hbm, dst_hbm, g_hbm,
                   sidx, didx, rows, acc, gsem, isem, ssem):
    for b in range(min(NIDX, nch)):
        pltpu.async_copy(src_hbm.at[base + b], sidx[b], isem[b])
        pltpu.async_copy(dst_hbm.at[base + b], didx[b], isem[b])

    def wait_idx(i, b):
        pltpu.make_async_copy(src_hbm.at[base + i], sidx[b], isem[b]).wait()
        pltpu.make_async_copy(dst_hbm.at[base + i], didx[b], isem[b]).wait()

    for k in range(min(NROW, nch)):
        wait_idx(k, k % NIDX)
        pltpu.async_copy(g_hbm.at[sidx[k % NIDX]], rows[k % NROW], gsem[k % NROW])

    def outer(g, _):
        for b in range(NIDX):
            i = g * NIDX + b
            rb = b % NROW
            pltpu.make_async_copy(g_hbm.at[sidx[b]], rows[rb],
                                  gsem[rb]).wait()
            pltpu.async_copy(rows[rb], acc.at[didx[b]], ssem,
                             add=True).wait()
            if nch > NIDX:
                @pl.when(i + NIDX < nch)
                def _():
                    pltpu.async_copy(src_hbm.at[base + i + NIDX], sidx[b],
                                     isem[b])
                    pltpu.async_copy(dst_hbm.at[base + i + NIDX], didx[b],
                                     isem[b])
            if nch > NROW:
                @pl.when(i + NROW < nch)
                def _():
                    b2 = (b + NROW) % NIDX
                    wait_idx(i + NROW, b2)
                    pltpu.async_copy(g_hbm.at[sidx[b2]], rows[rb],
                                     gsem[rb])
        return ()

    lax.fori_loop(0, nch // NIDX, outer, ())


@functools.partial(
    pl.kernel,
    out_type=jax.ShapeDtypeStruct((NC, NPAD, F), jnp.float32),
    mesh=_mesh,
    scratch_types=[
        [pltpu.VMEM((CH,), jnp.int32) for _ in range(NIDX)],
        [pltpu.VMEM((CH,), jnp.int32) for _ in range(NIDX)],
        [pltpu.VMEM((CH, F), jnp.float32) for _ in range(NROW)],
        pltpu.VMEM_SHARED((NPAD, F), jnp.float32),
        [pltpu.SemaphoreType.DMA for _ in range(NROW)],
        [pltpu.SemaphoreType.DMA for _ in range(NIDX)],
        pltpu.SemaphoreType.DMA,
    ],
)
def _scatter_sc(src_hbm, dst_hbm, g_hbm, zeros_hbm, out_hbm,
                sidx, didx, rows, acc, gsem, isem, ssem):
    c = lax.axis_index("c")
    s = lax.axis_index("s")
    pltpu.sync_copy(zeros_hbm, acc.at[pl.ds(s * ROWS_T, ROWS_T)])
    plsc.subcore_barrier()

    @pl.when(c == 0)
    def _():
        _edge_pipeline(NCH_C0, s * NCH_C0, src_hbm, dst_hbm, g_hbm,
                       sidx, didx, rows, acc, gsem, isem, ssem)

    @pl.when(c == 1)
    def _():
        _edge_pipeline(NCH_C1, NS * NCH_C0 + s * NCH_C1, src_hbm, dst_hbm,
                       g_hbm, sidx, didx, rows, acc, gsem, isem, ssem)

    plsc.subcore_barrier()
    pltpu.sync_copy(acc.at[pl.ds(s * ROWS_T, ROWS_T)],
                    out_hbm.at[c, pl.ds(s * ROWS_T, ROWS_T)])


def _dinv(deg0_ref, deg1_ref):
    deg = deg0_ref[...] + deg1_ref[...] + 1.0
    return lax.rsqrt(deg)


def _first_tc_body(x_ref, deg0_ref, deg1_ref, w_ref, g_ref):
    h = jnp.dot(x_ref[...], w_ref[...], preferred_element_type=jnp.float32)
    g_ref[...] = _dinv(deg0_ref, deg1_ref) * h


def _mid_tc_body(a0_ref, a1_ref, gp_ref, deg0_ref, deg1_ref, b_ref, w_ref,
                 g_ref):
    dinv = _dinv(deg0_ref, deg1_ref)
    y = jnp.maximum(dinv * (a0_ref[...] + a1_ref[...] + gp_ref[...])
                    + b_ref[...], 0.0)
    g_ref[...] = dinv * jnp.dot(y, w_ref[...],
                                preferred_element_type=jnp.float32)


def _final_tc_body(a0_ref, a1_ref, gp_ref, deg0_ref, deg1_ref, b_ref,
                   batch_ref, wout_ref, bout_ref, out_ref, sums, counts):
    i = pl.program_id(0)

    @pl.when(i == 0)
    def _():
        sums[...] = jnp.zeros_like(sums)
        counts[...] = jnp.zeros_like(counts)

    dinv = _dinv(deg0_ref, deg1_ref)
    y = jnp.maximum(dinv * (a0_ref[...] + a1_ref[...] + gp_ref[...])
                    + b_ref[...], 0.0)
    oh = (jnp.broadcast_to(batch_ref[...].reshape(1, BR), (G, BR))
          == lax.broadcasted_iota(jnp.int32, (G, BR), 0)).astype(jnp.float32)
    sums[...] += jnp.dot(oh, y, preferred_element_type=jnp.float32)
    counts[...] += jnp.broadcast_to(
        jnp.sum(oh, axis=1, keepdims=True), (G, F))

    @pl.when(i == pl.num_programs(0) - 1)
    def _():
        pooled = sums[...] / jnp.maximum(counts[...], 1.0)
        out_ref[...] = (jnp.dot(pooled, wout_ref[...],
                                preferred_element_type=jnp.float32)
                        + bout_ref[...])


def _row_spec():
    return pl.BlockSpec((BR, F), lambda i: (i, 0))


def _deg_spec():
    return pl.BlockSpec((BR, 1), lambda i: (i, 0))


def _full_spec(r, c):
    return pl.BlockSpec((r, c), lambda i: (0, 0))


def _first_tc(xp, deg0, deg1, w):
    return pl.pallas_call(
        _first_tc_body,
        grid=(NBLK,),
        in_specs=[_row_spec(), _deg_spec(), _deg_spec(), _full_spec(F, F)],
        out_specs=_row_spec(),
        out_shape=jax.ShapeDtypeStruct((NPAD, F), jnp.float32),
    )(xp, deg0, deg1, w)


def _mid_tc(a0, a1, gp, deg0, deg1, b2d, w):
    return pl.pallas_call(
        _mid_tc_body,
        grid=(NBLK,),
        in_specs=[_row_spec(), _row_spec(), _row_spec(), _deg_spec(),
                  _deg_spec(), _full_spec(1, F), _full_spec(F, F)],
        out_specs=_row_spec(),
        out_shape=jax.ShapeDtypeStruct((NPAD, F), jnp.float32),
    )(a0, a1, gp, deg0, deg1, b2d, w)


def _final_tc(a0, a1, gp, deg0, deg1, b2d, batch2d, wout, bout2d):
    return pl.pallas_call(
        _final_tc_body,
        grid=(NBLK,),
        in_specs=[_row_spec(), _row_spec(), _row_spec(), _deg_spec(),
                  _deg_spec(), _full_spec(1, F),
                  pl.BlockSpec((1, 1, BR), lambda i: (i, 0, 0)),
                  _full_spec(F, F), _full_spec(1, F)],
        out_specs=_full_spec(G, F),
        out_shape=jax.ShapeDtypeStruct((G, F), jnp.float32),
        scratch_shapes=[pltpu.VMEM((G, F), jnp.float32),
                        pltpu.VMEM((G, F), jnp.float32)],
    )(a0, a1, gp, deg0, deg1, b2d, batch2d, wout, bout2d)


def kernel(x, edge_index, batch, W1, b1, W2, b2, W3, b3, Wout, bout):
    f32 = jnp.float32
    xp = jnp.concatenate([x, jnp.zeros((NPAD - N, F), f32)], axis=0)
    pad_e = jnp.full((EPAD - E,), NPAD - 1, jnp.int32)
    dst = jnp.concatenate([edge_index[1], pad_e])
    src2d = jnp.concatenate([edge_index[0], pad_e]).reshape(NW * NCHUNK, CH)
    dst2d = dst.reshape(NW * NCHUNK, CH)
    batch2d = jnp.concatenate(
        [batch.astype(jnp.int32), jnp.full((NPAD - N,), G, jnp.int32)]
    ).reshape(NBLK, 1, BR)

    zeros1 = jnp.zeros((ROWS_T,), f32)
    zeros128 = jnp.zeros((ROWS_T, F), f32)
    ones1 = jnp.ones((CH,), f32)

    degs = _deg_sc(dst, ones1, zeros1)
    deg0 = degs[:NPAD].reshape(NPAD, 1)
    deg1 = degs[NPAD:].reshape(NPAD, 1)

    b1r = b1.reshape(1, F)
    b2r = b2.reshape(1, F)
    b3r = b3.reshape(1, F)
    boutr = bout.reshape(1, F)

    g1 = _first_tc(xp, deg0, deg1, W1)
    a1 = _scatter_sc(src2d, dst2d, g1, zeros128)
    g2 = _mid_tc(a1[0], a1[1], g1, deg0, deg1, b1r, W2)
    a2 = _scatter_sc(src2d, dst2d, g2, zeros128)
    g3 = _mid_tc(a2[0], a2[1], g2, deg0, deg1, b2r, W3)
    a3 = _scatter_sc(src2d, dst2d, g3, zeros128)
    return _final_tc(a3[0], a3[1], g3, deg0, deg1, b3r, batch2d, Wout, boutr)
